# baseline (speedup 1.0000x reference)
import functools

import jax
import jax.numpy as jnp
from jax import lax
from jax.experimental import pallas as pl
from jax.experimental.pallas import tpu as pltpu
from jax.experimental.pallas import tpu_sc as plsc

NC, NS, L = 2, 16, 16
NW = NC * NS
BATCH, SEQ, HIDDEN = 1024, 512, 128
NTOK = BATCH * SEQ
TPW = NTOK // NW
C = 64
CPSS = 32
SCTOK = C * CPSS
NSS = TPW // SCTOK
NJ = HIDDEN // L


def _rsqrt16(v):
    i = lax.bitcast_convert_type(v, jnp.int32)
    y = lax.bitcast_convert_type(jnp.int32(0x5F3759DF) - (i >> 1), jnp.float32)
    for _ in range(2):
        y = y * (1.5 - 0.5 * v * y * y)
    return y


def _body(ids_hbm, tt_hbm, tok_hbm, pos_hbm, type_hbm, gam_hbm, bet_hbm,
          out_hbm, idx_sc, tt_sc, rows0, rows1, out0, out1, pos_v, type_v,
          gam_v, bet_v, sem_g, sem_o):
    rows = (rows0, rows1)
    outs = (out0, out1)
    wid = lax.axis_index("s") * NC + lax.axis_index("c")

    pltpu.sync_copy(pos_hbm, pos_v)
    pltpu.sync_copy(type_hbm, type_v)
    pltpu.sync_copy(gam_hbm, gam_v)
    pltpu.sync_copy(bet_hbm, bet_v)

    def gather_start(k, buf):
        pltpu.async_copy(tok_hbm.at[idx_sc.at[pl.ds(k * C, C)]], buf, sem_g)

    def gather_wait(buf):
        pltpu.make_async_copy(
            tok_hbm.at[idx_sc.at[pl.ds(0, C)]], buf, sem_g).wait()

    def out_wait(buf):
        pltpu.make_async_copy(buf, out_hbm.at[pl.ds(0, C)], sem_o).wait()

    def compute(p0, toff, rbuf, obuf):
        @plsc.parallel_loop(0, C, unroll=6)
        def _tok(i):
            tt = tt_sc[pl.ds(toff + i, L)][0]
            p = p0 + i
            xs = []
            for j in range(NJ):
                sl = pl.ds(j * L, L)
                xs.append(rbuf[i, sl] + pos_v[p, sl] + type_v[tt, sl])

            def tree(vs):
                while len(vs) > 1:
                    vs = [a + b for a, b in zip(vs[::2], vs[1::2])]
                return vs[0]

            s = tree(xs)
            sq = tree([x * x for x in xs])
            tot = jnp.sum(s)
            totsq = jnp.sum(sq)
            meanv = lax.broadcast(tot, (L,)) * (1.0 / HIDDEN)
            varv = lax.broadcast(totsq, (L,)) * (1.0 / HIDDEN) \
                - meanv * meanv + 1e-5
            rstd = _rsqrt16(varv)
            for j in range(NJ):
                sl = pl.ds(j * L, L)
                obuf[i, sl] = (xs[j] - meanv) * rstd

    @pl.loop(0, NSS)
    def _ss(s):
        ids_off = wid * TPW + s * SCTOK
        pltpu.sync_copy(ids_hbm.at[pl.ds(ids_off, SCTOK)], idx_sc)
        pltpu.sync_copy(tt_hbm.at[pl.ds(ids_off, SCTOK)],
                        tt_sc.at[pl.ds(0, SCTOK)])
        gather_start(0, rows[0])

        @pl.loop(0, CPSS, step=2)
        def _chunk(k0):
            for b in range(2):
                k = k0 + b
                gci = s * CPSS + k
                gather_wait(rows[b])

                @pl.when(k + 1 < CPSS)
                def _():
                    gather_start(k + 1, rows[1 - b])

                @pl.when(gci >= 2)
                def _():
                    out_wait(outs[b])

                p0 = (gci % (SEQ // C)) * C
                compute(p0, k * C, rows[b], outs[b])
                pltpu.async_copy(
                    outs[b], out_hbm.at[pl.ds(wid * TPW + gci * C, C)],
                    sem_o)

    out_wait(outs[0])
    out_wait(outs[1])


@jax.jit
def _run(ids, tts, tok_emb, pos_emb, type_emb, gamma, beta):
    mesh = plsc.VectorSubcoreMesh(core_axis_name="c", subcore_axis_name="s",
                                  num_cores=NC, num_subcores=NS)
    f = pl.kernel(
        _body,
        out_type=jax.ShapeDtypeStruct((NTOK, HIDDEN), jnp.float32),
        mesh=mesh,
        compiler_params=pltpu.CompilerParams(needs_layout_passes=False),
        scratch_types=[
            pltpu.VMEM((SCTOK,), jnp.int32),
            pltpu.VMEM((SCTOK + L,), jnp.int32),
            pltpu.VMEM((C, HIDDEN), jnp.float32),
            pltpu.VMEM((C, HIDDEN), jnp.float32),
            pltpu.VMEM((C, HIDDEN), jnp.float32),
            pltpu.VMEM((C, HIDDEN), jnp.float32),
            pltpu.VMEM((SEQ, HIDDEN), jnp.float32),
            pltpu.VMEM((3, HIDDEN), jnp.float32),
            pltpu.VMEM((HIDDEN,), jnp.float32),
            pltpu.VMEM((HIDDEN,), jnp.float32),
            pltpu.SemaphoreType.DMA,
            pltpu.SemaphoreType.DMA,
        ],
    )
    return f(ids, tts, tok_emb, pos_emb, type_emb, gamma, beta)


def kernel(input_ids, token_type_ids, tok_emb, pos_emb, type_emb, gamma, beta):
    ids = input_ids.reshape(-1).astype(jnp.int32)
    tts = token_type_ids.reshape(-1).astype(jnp.int32)
    out = _run(ids, tts, tok_emb, pos_emb, type_emb, gamma, beta)
    return out.reshape(BATCH, SEQ, HIDDEN)


# speedup vs baseline: 1.9291x; 1.9291x over previous
import functools

import jax
import jax.numpy as jnp
from jax import lax
from jax.experimental import pallas as pl
from jax.experimental.pallas import tpu as pltpu
from jax.experimental.pallas import tpu_sc as plsc

NC, NS, L = 2, 16, 16
NW = NC * NS
BATCH, SEQ, HIDDEN = 1024, 512, 128
NTOK = BATCH * SEQ
TPW = NTOK // NW
C = 64
CPSS = 32
SCTOK = C * CPSS
NSS = TPW // SCTOK
NJ = HIDDEN // L


def _rsqrt16(v):
    i = lax.bitcast_convert_type(v, jnp.int32)
    y = lax.bitcast_convert_type(jnp.int32(0x5F3759DF) - (i >> 1), jnp.float32)
    for _ in range(2):
        y = y * (1.5 - 0.5 * v * y * y)
    return y


def _body(ids_hbm, tt_hbm, tok_hbm, pos_hbm, type_hbm, gam_hbm, bet_hbm,
          out_hbm, idx_sc, tt_sc, rows0, rows1, out0, out1, pos_v, type_v,
          gam_v, bet_v, sem_g, sem_o):
    rows = (rows0, rows1)
    outs = (out0, out1)
    wid = lax.axis_index("s") * NC + lax.axis_index("c")

    pltpu.sync_copy(pos_hbm, pos_v)
    pltpu.sync_copy(type_hbm, type_v)
    pltpu.sync_copy(gam_hbm, gam_v)
    pltpu.sync_copy(bet_hbm, bet_v)

    def gather_start(k, buf):
        pltpu.async_copy(tok_hbm.at[idx_sc.at[pl.ds(k * C, C)]], buf, sem_g)

    def gather_wait(buf):
        pltpu.make_async_copy(
            tok_hbm.at[idx_sc.at[pl.ds(0, C)]], buf, sem_g).wait()

    def out_wait(buf):
        pltpu.make_async_copy(buf, out_hbm.at[pl.ds(0, C)], sem_o).wait()

    def compute(p0, toff, rbuf, obuf):
        @plsc.parallel_loop(0, C, unroll=8)
        def _tok(i):
            tt = tt_sc[pl.ds(toff + i, L)][0]
            p = p0 + i
            xs = []
            for j in range(NJ):
                sl = pl.ds(j * L, L)
                xs.append(rbuf[i, sl] + pos_v[p, sl] + type_v[tt, sl])

            def tree(vs):
                while len(vs) > 1:
                    vs = [a + b for a, b in zip(vs[::2], vs[1::2])]
                return vs[0]

            s = tree(xs)
            sq = tree([x * x for x in xs])
            tot = jnp.sum(s)
            totsq = jnp.sum(sq)
            meanv = lax.broadcast(tot, (L,)) * (1.0 / HIDDEN)
            varv = lax.broadcast(totsq, (L,)) * (1.0 / HIDDEN) \
                - meanv * meanv + 1e-5
            rstd = _rsqrt16(varv)
            for j in range(NJ):
                sl = pl.ds(j * L, L)
                obuf[i, sl] = (xs[j] - meanv) * rstd

    @pl.loop(0, NSS)
    def _ss(s):
        ids_off = wid * TPW + s * SCTOK
        pltpu.sync_copy(ids_hbm.at[pl.ds(ids_off, SCTOK)], idx_sc)
        pltpu.sync_copy(tt_hbm.at[pl.ds(ids_off, SCTOK)],
                        tt_sc.at[pl.ds(0, SCTOK)])
        gather_start(0, rows[0])

        @pl.loop(0, CPSS, step=2)
        def _chunk(k0):
            for b in range(2):
                k = k0 + b
                gci = s * CPSS + k
                gather_wait(rows[b])

                @pl.when(k + 1 < CPSS)
                def _():
                    gather_start(k + 1, rows[1 - b])

                @pl.when(gci >= 2)
                def _():
                    out_wait(outs[b])

                p0 = (gci % (SEQ // C)) * C
                compute(p0, k * C, rows[b], outs[b])
                pltpu.async_copy(
                    outs[b], out_hbm.at[pl.ds(wid * TPW + gci * C, C)],
                    sem_o)

    out_wait(outs[0])
    out_wait(outs[1])


@jax.jit
def _run(ids, tts, tok_emb, pos_emb, type_emb, gamma, beta):
    mesh = plsc.VectorSubcoreMesh(core_axis_name="c", subcore_axis_name="s",
                                  num_cores=NC, num_subcores=NS)
    f = pl.kernel(
        _body,
        out_type=jax.ShapeDtypeStruct((NTOK, HIDDEN), jnp.float32),
        mesh=mesh,
        compiler_params=pltpu.CompilerParams(needs_layout_passes=False),
        scratch_types=[
            pltpu.VMEM((SCTOK,), jnp.int32),
            pltpu.VMEM((SCTOK + L,), jnp.int32),
            pltpu.VMEM((C, HIDDEN), jnp.float32),
            pltpu.VMEM((C, HIDDEN), jnp.float32),
            pltpu.VMEM((C, HIDDEN), jnp.float32),
            pltpu.VMEM((C, HIDDEN), jnp.float32),
            pltpu.VMEM((SEQ, HIDDEN), jnp.float32),
            pltpu.VMEM((3, HIDDEN), jnp.float32),
            pltpu.VMEM((HIDDEN,), jnp.float32),
            pltpu.VMEM((HIDDEN,), jnp.float32),
            pltpu.SemaphoreType.DMA,
            pltpu.SemaphoreType.DMA,
        ],
    )
    return f(ids, tts, tok_emb, pos_emb, type_emb, gamma, beta)


def kernel(input_ids, token_type_ids, tok_emb, pos_emb, type_emb, gamma, beta):
    ids = input_ids.reshape(-1).astype(jnp.int32)
    tts = token_type_ids.reshape(-1).astype(jnp.int32)
    out = _run(ids, tts, tok_emb, pos_emb, type_emb, gamma, beta)
    return out.reshape(BATCH, SEQ, HIDDEN)


# speedup vs baseline: 2.0264x; 1.0504x over previous
import functools

import jax
import jax.numpy as jnp
from jax import lax
from jax.experimental import pallas as pl
from jax.experimental.pallas import tpu as pltpu
from jax.experimental.pallas import tpu_sc as plsc

NC, NS, L = 2, 16, 16
NW = NC * NS
BATCH, SEQ, HIDDEN = 1024, 512, 128
NTOK = BATCH * SEQ
TPW = NTOK // NW
C = 64
CPSS = 32
SCTOK = C * CPSS
NSS = TPW // SCTOK
NJ = HIDDEN // L


def _rsqrt16(v):
    i = lax.bitcast_convert_type(v, jnp.int32)
    y = lax.bitcast_convert_type(jnp.int32(0x5F3759DF) - (i >> 1), jnp.float32)
    for _ in range(2):
        y = y * (1.5 - 0.5 * v * y * y)
    return y


def _body(ids_hbm, tt_hbm, tok_hbm, pos_hbm, type_hbm, gam_hbm, bet_hbm,
          out_hbm, idx_a, idx_b, tt_a, tt_b, rows0, rows1, out0, out1,
          pos_v, type_v, sem_g, sem_o, sem_i):
    rows = (rows0, rows1)
    outs = (out0, out1)
    wid = lax.axis_index("s") * NC + lax.axis_index("c")

    pltpu.sync_copy(pos_hbm, pos_v)
    pltpu.sync_copy(type_hbm, type_v)

    def ids_start(ss, idx_buf, tt_buf):
        off = wid * TPW + ss * SCTOK
        pltpu.async_copy(ids_hbm.at[pl.ds(off, SCTOK)], idx_buf, sem_i)
        pltpu.async_copy(tt_hbm.at[pl.ds(off, SCTOK)],
                         tt_buf.at[pl.ds(0, SCTOK)], sem_i)

    def ids_wait():
        pltpu.make_async_copy(
            ids_hbm.at[pl.ds(0, SCTOK)], idx_a, sem_i).wait()
        pltpu.make_async_copy(
            ids_hbm.at[pl.ds(0, SCTOK)], idx_a, sem_i).wait()

    def gather_start(idx_buf, k, buf):
        pltpu.async_copy(tok_hbm.at[idx_buf.at[pl.ds(k * C, C)]], buf, sem_g)

    def gather_wait(buf):
        pltpu.make_async_copy(
            tok_hbm.at[idx_a.at[pl.ds(0, C)]], buf, sem_g).wait()

    def out_wait(buf):
        pltpu.make_async_copy(buf, out_hbm.at[pl.ds(0, C)], sem_o).wait()

    def compute(p0, toff, tt_buf, rbuf, obuf):
        @plsc.parallel_loop(0, C, unroll=8)
        def _tok(i):
            tt = tt_buf[pl.ds(toff + i, L)][0]
            p = p0 + i
            xs = []
            for j in range(NJ):
                sl = pl.ds(j * L, L)
                xs.append(rbuf[i, sl] + pos_v[p, sl] + type_v[tt, sl])

            def tree(vs):
                while len(vs) > 1:
                    vs = [a + b for a, b in zip(vs[::2], vs[1::2])]
                return vs[0]

            s = tree(xs)
            sq = tree([x * x for x in xs])
            tot = jnp.sum(s)
            totsq = jnp.sum(sq)
            meanv = lax.broadcast(tot, (L,)) * (1.0 / HIDDEN)
            varv = lax.broadcast(totsq, (L,)) * (1.0 / HIDDEN) \
                - meanv * meanv + 1e-5
            rstd = _rsqrt16(varv)
            for j in range(NJ):
                sl = pl.ds(j * L, L)
                obuf[i, sl] = (xs[j] - meanv) * rstd

    pltpu.sync_copy(ids_hbm.at[pl.ds(wid * TPW, SCTOK)], idx_a)
    pltpu.sync_copy(tt_hbm.at[pl.ds(wid * TPW, SCTOK)],
                    tt_a.at[pl.ds(0, SCTOK)])
    ids_start(1, idx_b, tt_b)
    gather_start(idx_a, 0, rows[0])

    @pl.loop(0, NSS, step=2)
    def _ss(s0):
        for sb in range(2):
            s = s0 + sb
            idx_cur, tt_cur = (idx_a, tt_a) if sb == 0 else (idx_b, tt_b)
            idx_nxt, tt_nxt = (idx_b, tt_b) if sb == 0 else (idx_a, tt_a)

            @pl.loop(0, CPSS, step=2)
            def _chunk(k0):
                for b in range(2):
                    k = k0 + b
                    gci = s * CPSS + k
                    gather_wait(rows[b])

                    @pl.when(k + 1 < CPSS)
                    def _():
                        gather_start(idx_cur, k + 1, rows[1 - b])

                    @pl.when((k + 1 == CPSS) & (s + 1 < NSS))
                    def _():
                        ids_wait()
                        gather_start(idx_nxt, 0, rows[1 - b])

                    @pl.when(gci >= 2)
                    def _():
                        out_wait(outs[b])

                    p0 = (gci % (SEQ // C)) * C
                    compute(p0, k * C, tt_cur, rows[b], outs[b])
                    pltpu.async_copy(
                        outs[b], out_hbm.at[pl.ds(wid * TPW + gci * C, C)],
                        sem_o)

            @pl.when(s + 2 < NSS)
            def _():
                ids_start(s + 2, idx_cur, tt_cur)

    out_wait(outs[0])
    out_wait(outs[1])


@jax.jit
def _run(ids, tts, tok_emb, pos_emb, type_emb, gamma, beta):
    mesh = plsc.VectorSubcoreMesh(core_axis_name="c", subcore_axis_name="s",
                                  num_cores=NC, num_subcores=NS)
    f = pl.kernel(
        _body,
        out_type=jax.ShapeDtypeStruct((NTOK, HIDDEN), jnp.float32),
        mesh=mesh,
        compiler_params=pltpu.CompilerParams(needs_layout_passes=False),
        scratch_types=[
            pltpu.VMEM((SCTOK,), jnp.int32),
            pltpu.VMEM((SCTOK,), jnp.int32),
            pltpu.VMEM((SCTOK + L,), jnp.int32),
            pltpu.VMEM((SCTOK + L,), jnp.int32),
            pltpu.VMEM((C, HIDDEN), jnp.float32),
            pltpu.VMEM((C, HIDDEN), jnp.float32),
            pltpu.VMEM((C, HIDDEN), jnp.float32),
            pltpu.VMEM((C, HIDDEN), jnp.float32),
            pltpu.VMEM((SEQ, HIDDEN), jnp.float32),
            pltpu.VMEM((3, HIDDEN), jnp.float32),
            pltpu.SemaphoreType.DMA,
            pltpu.SemaphoreType.DMA,
            pltpu.SemaphoreType.DMA,
        ],
    )
    return f(ids, tts, tok_emb, pos_emb, type_emb, gamma, beta)


def kernel(input_ids, token_type_ids, tok_emb, pos_emb, type_emb, gamma, beta):
    ids = input_ids.reshape(-1).astype(jnp.int32)
    tts = token_type_ids.reshape(-1).astype(jnp.int32)
    out = _run(ids, tts, tok_emb, pos_emb, type_emb, gamma, beta)
    return out.reshape(BATCH, SEQ, HIDDEN)
